# baseline (speedup 1.0000x reference)
import jax
import jax.numpy as jnp
import numpy as np
from jax.experimental import pallas as pl
from jax.experimental.pallas import tpu as pltpu

H_P, W_P = 14, 14
N_TOK = H_P * W_P
EMBED_DIM = 768
NUM_HEADS = 12
DEPTH = 12
HIDDEN_DIM = 3072
R_C, R_F = 4, 8
TAU = 1.0
EPS = 1e-6
ALPHA, BETA, GAMMA = 1.0, 0.5, 0.5
RHO = 0.5
TOK_TEMP = 0.1
B, T = 8, 16
K_TOP = max(1, int(RHO * T * N_TOK))

TILE = 8
N_FULL = N_TOK // TILE
REM = N_TOK - N_FULL * TILE
N_TILES = N_FULL + 1
N_PAD = N_TILES * TILE


def _interp_coeffs(t_in, t_out):
    src = ((np.arange(t_out, dtype=np.float32) + np.float32(0.5))
           * np.float32(t_in / float(t_out)) - np.float32(0.5))
    src = np.clip(src, np.float32(0.0), np.float32(t_in - 1.0))
    lo = np.floor(src).astype(np.int32)
    hi = np.minimum(lo + 1, t_in - 1)
    w = (src - lo.astype(np.float32)).astype(np.float32)
    return lo, hi, w


I2_COEF = _interp_coeffs(T - 1, T)
I4_COEF = _interp_coeffs(T - 3, T)


def _interp_t(e, coef):
    lo, hi, w = coef
    rows = []
    for t in range(T):
        w1 = float(np.float32(1.0) - w[t])
        rows.append(e[:, int(lo[t]), :] * w1 + e[:, int(hi[t]), :] * float(w[t]))
    return jnp.stack(rows, axis=1)


def _region_one_hot(coords, centers):
    d = jnp.sqrt(jnp.maximum(
        ((coords[:, None, :] - centers[None, :, :]) ** 2).sum(-1), 0.0))
    rid = jnp.argmin(d, axis=1)
    return (rid[None, :] == jnp.arange(centers.shape[0])[:, None]).astype(
        jnp.float32)


def _kth_largest(score, k):
    hi0 = jnp.max(score) + 1.0
    lo0 = jnp.zeros((), jnp.float32)

    def body(_, carry):
        lo, hi = carry
        mid = 0.5 * (lo + hi)
        cnt = jnp.sum((score >= mid).astype(jnp.float32))
        ge = cnt >= float(k)
        return jnp.where(ge, mid, lo), jnp.where(ge, hi, mid)

    lo, _ = jax.lax.fori_loop(0, 32, body, (lo0, hi0))
    return lo


def _fused_kernel(x_ref, oh_ref, ohp_ref, vm_ref,
                  mask_ref, psum_ref,
                  e1_scr, e2_scr, e4_scr, hr_scr, p_scr):
    oh = oh_ref[...]
    cnt = jnp.sum(oh, axis=1, keepdims=True)

    def _h(q):
        return -jnp.sum(q * jnp.log(q + EPS), axis=1)

    def do_tile(row_slc, tile_idx, h):
        s_hist = []
        c1, c2, c4 = [], [], []
        for t in range(T):
            x = x_ref[0, t, row_slc, :]
            m = jnp.max(x, axis=1, keepdims=True)
            e = jnp.exp(x - m)
            z = jnp.sum(e, axis=1, keepdims=True)
            p = e / z
            p_scr[t, row_slc, :] = p

            s_t = p if t == 0 else s_hist[t - 1] + p
            q1 = p if t == 0 else s_t - s_hist[t - 1]
            c1.append(_h(q1))
            if t >= 1:
                q2 = s_t * 0.5 if t == 1 else (s_t - s_hist[t - 2]) * 0.5
                c2.append(_h(q2))
            if t >= 3:
                q4 = s_t * 0.25 if t == 3 else (s_t - s_hist[t - 4]) * 0.25
                c4.append(_h(q4))
            s_hist.append(s_t)
        e1_scr[tile_idx, :, 0:h] = jnp.stack(c1, axis=0)
        e2_scr[tile_idx, 1:T, 0:h] = jnp.stack(c2, axis=0)
        e4_scr[tile_idx, 3:T, 0:h] = jnp.stack(c4, axis=0)

    def tile_body(i, carry):
        do_tile(pl.ds(i * TILE, TILE), i, TILE)
        return carry

    jax.lax.fori_loop(0, N_FULL, tile_body, 0)
    do_tile(slice(N_FULL * TILE, N_TOK), N_FULL, REM)

    for t in range(T):
        p_sum = jnp.dot(oh, p_scr[t], preferred_element_type=jnp.float32)
        p_reg = p_sum / (cnt + EPS)
        hr_scr[t, :] = _h(p_reg)

    vm = vm_ref[...] > 0.5
    vm3 = jnp.broadcast_to(vm, (N_TILES, T, TILE))
    e1 = e1_scr[...]
    i2 = _interp_t(e2_scr[:, 1:T, :], I2_COEF)
    i4 = _interp_t(e4_scr[:, 3:T, :], I4_COEF)
    ht = (e1 + i2 + i4) * (1.0 / 3.0)
    mn = jnp.min(jnp.where(vm3, ht, 1e30))
    mx = jnp.max(jnp.where(vm3, ht, -1e30))
    ht_n = (ht - mn) / (mx - mn + EPS)
    hr = hr_scr[...]

    def _nrm(hsub):
        return (hsub - jnp.min(hsub)) / (jnp.max(hsub) - jnp.min(hsub) + EPS)

    hc_n = _nrm(hr[:, :R_C])
    hf_n = _nrm(hr[:, R_C:])
    ohp = ohp_ref[...]
    hc_tok = sum(hc_n[:, r].reshape(1, T, 1) * ohp[r][:, None, :]
                 for r in range(R_C))
    hf_tok = sum(hf_n[:, r].reshape(1, T, 1) * ohp[R_C + r][:, None, :]
                 for r in range(R_F))
    score = ALPHA * ht_n + BETA * hc_tok + GAMMA * hf_tok
    score_m = jnp.where(vm3, score, -1.0)
    kth = _kth_largest(score_m, K_TOP)
    mask = jax.nn.sigmoid((score - kth) * (1.0 / TOK_TEMP))
    mask_ref[0] = mask
    psum_ref[...] = jnp.reshape(jnp.sum(jnp.where(vm3, mask, 0.0)), (1, 1, 1))


def _scalar_kernel(psum_ref, ghead_ref, gch_ref, gblock_ref,
                   headw_ref, chw_ref, blockw_ref, st_ref, last_ref):
    head_w = jax.nn.sigmoid(ghead_ref[...])
    ch_w = jax.nn.sigmoid(gch_ref[...])
    block_w = jax.nn.sigmoid(gblock_ref[...])
    headw_ref[...] = head_w
    chw_ref[...] = ch_w
    blockw_ref[...] = block_w
    total = jnp.sum(psum_ref[...])
    sparsity_token = 1.0 - total / float(B * T * N_TOK)
    l_ast = (sparsity_token + (1.0 - jnp.mean(head_w))
             + (1.0 - jnp.mean(ch_w)) + (1.0 - jnp.mean(block_w)))
    st_ref[...] = jnp.reshape(sparsity_token, (1, 1))
    last_ref[...] = jnp.reshape(l_ast, (1, 1))


def kernel(token_feat, centers_coarse, centers_fine, g_head, g_ch, g_block,
           patch_coords):
    oh = jnp.concatenate([
        _region_one_hot(patch_coords, centers_coarse),
        _region_one_hot(patch_coords, centers_fine),
    ], axis=0)
    oh_pad = jnp.pad(oh, ((0, 0), (0, N_PAD - N_TOK))).reshape(
        R_C + R_F, N_TILES, TILE)
    vmask = (np.arange(N_PAD).reshape(N_TILES, 1, TILE) < N_TOK
             ).astype(np.float32)

    n, c = N_TOK, EMBED_DIM
    const = lambda b: (0, 0)
    mask4, psum = pl.pallas_call(
        _fused_kernel,
        grid=(B,),
        in_specs=[
            pl.BlockSpec((1, T, n, c), lambda b: (b, 0, 0, 0)),
            pl.BlockSpec((R_C + R_F, n), const),
            pl.BlockSpec((R_C + R_F, N_TILES, TILE), lambda b: (0, 0, 0)),
            pl.BlockSpec((N_TILES, 1, TILE), lambda b: (0, 0, 0)),
        ],
        out_specs=[
            pl.BlockSpec((1, N_TILES, T, TILE), lambda b: (b, 0, 0, 0)),
            pl.BlockSpec((1, 1, 1), lambda b: (b, 0, 0)),
        ],
        out_shape=[
            jax.ShapeDtypeStruct((B, N_TILES, T, TILE), jnp.float32),
            jax.ShapeDtypeStruct((B, 1, 1), jnp.float32),
        ],
        scratch_shapes=[
            pltpu.VMEM((N_TILES, T, TILE), jnp.float32),
            pltpu.VMEM((N_TILES, T, TILE), jnp.float32),
            pltpu.VMEM((N_TILES, T, TILE), jnp.float32),
            pltpu.VMEM((T, R_C + R_F), jnp.float32),
            pltpu.VMEM((T, n, c), jnp.float32),
        ],
        compiler_params=pltpu.CompilerParams(
            dimension_semantics=("parallel",)),
    )(token_feat, oh, oh_pad, jnp.asarray(vmask))

    head_w, ch_w, block_w2, st, last = pl.pallas_call(
        _scalar_kernel,
        out_shape=[
            jax.ShapeDtypeStruct((DEPTH, NUM_HEADS), jnp.float32),
            jax.ShapeDtypeStruct((DEPTH, HIDDEN_DIM), jnp.float32),
            jax.ShapeDtypeStruct((1, DEPTH), jnp.float32),
            jax.ShapeDtypeStruct((1, 1), jnp.float32),
            jax.ShapeDtypeStruct((1, 1), jnp.float32),
        ],
    )(psum, g_head, g_ch, g_block.reshape(1, DEPTH))
    mask = jnp.transpose(mask4, (0, 2, 1, 3)).reshape(B, T, N_PAD)[:, :, :n]
    return (mask, head_w, ch_w, block_w2.reshape(DEPTH),
            st.reshape(()), last.reshape(()))


# speedup vs baseline: 1.0056x; 1.0056x over previous
import jax
import jax.numpy as jnp
import numpy as np
from jax.experimental import pallas as pl
from jax.experimental.pallas import tpu as pltpu

H_P, W_P = 14, 14
N_TOK = H_P * W_P
EMBED_DIM = 768
NUM_HEADS = 12
DEPTH = 12
HIDDEN_DIM = 3072
R_C, R_F = 4, 8
TAU = 1.0
EPS = 1e-6
ALPHA, BETA, GAMMA = 1.0, 0.5, 0.5
RHO = 0.5
TOK_TEMP = 0.1
B, T = 8, 16
K_TOP = max(1, int(RHO * T * N_TOK))

TILE = 8
N_FULL = N_TOK // TILE
REM = N_TOK - N_FULL * TILE
N_TILES = N_FULL + 1
N_PAD = N_TILES * TILE


def _interp_coeffs(t_in, t_out):
    src = ((np.arange(t_out, dtype=np.float32) + np.float32(0.5))
           * np.float32(t_in / float(t_out)) - np.float32(0.5))
    src = np.clip(src, np.float32(0.0), np.float32(t_in - 1.0))
    lo = np.floor(src).astype(np.int32)
    hi = np.minimum(lo + 1, t_in - 1)
    w = (src - lo.astype(np.float32)).astype(np.float32)
    return lo, hi, w


I2_COEF = _interp_coeffs(T - 1, T)
I4_COEF = _interp_coeffs(T - 3, T)


def _interp_t(e, coef):
    lo, hi, w = coef
    rows = []
    for t in range(T):
        w1 = float(np.float32(1.0) - w[t])
        rows.append(e[:, int(lo[t]), :] * w1 + e[:, int(hi[t]), :] * float(w[t]))
    return jnp.stack(rows, axis=1)


def _region_one_hot(coords, centers):
    d = jnp.sqrt(jnp.maximum(
        ((coords[:, None, :] - centers[None, :, :]) ** 2).sum(-1), 0.0))
    rid = jnp.argmin(d, axis=1)
    return (rid[None, :] == jnp.arange(centers.shape[0])[:, None]).astype(
        jnp.float32)


def _kth_largest(score, k):
    hi0 = jnp.max(score) + 1.0
    lo0 = jnp.zeros((), jnp.float32)

    def body(_, carry):
        lo, hi = carry
        mid = 0.5 * (lo + hi)
        cnt = jnp.sum((score >= mid).astype(jnp.float32))
        ge = cnt >= float(k)
        return jnp.where(ge, mid, lo), jnp.where(ge, hi, mid)

    lo, _ = jax.lax.fori_loop(0, 32, body, (lo0, hi0))
    return lo


def _fused_kernel(x_ref, oh_ref, ohp_ref, vm_ref,
                  ghead_ref, gch_ref, gblock_ref,
                  mask_ref, headw_ref, chw_ref, blockw_ref, st_ref, last_ref,
                  e1_scr, e2_scr, e4_scr, hr_scr, tot_scr, p_scr):
    b = pl.program_id(0)
    oh = oh_ref[...]
    cnt = jnp.sum(oh, axis=1, keepdims=True)

    def _h(q):
        return -jnp.sum(q * jnp.log(q + EPS), axis=1)

    def do_tile(row_slc, tile_idx, h):
        s_hist = []
        c1, c2, c4 = [], [], []
        for t in range(T):
            x = x_ref[0, t, row_slc, :]
            m = jnp.max(x, axis=1, keepdims=True)
            e = jnp.exp(x - m)
            z = jnp.sum(e, axis=1, keepdims=True)
            p = e / z
            p_scr[t, row_slc, :] = p

            s_t = p if t == 0 else s_hist[t - 1] + p
            q1 = p if t == 0 else s_t - s_hist[t - 1]
            c1.append(_h(q1))
            if t >= 1:
                q2 = s_t * 0.5 if t == 1 else (s_t - s_hist[t - 2]) * 0.5
                c2.append(_h(q2))
            if t >= 3:
                q4 = s_t * 0.25 if t == 3 else (s_t - s_hist[t - 4]) * 0.25
                c4.append(_h(q4))
            s_hist.append(s_t)
        e1_scr[tile_idx, :, 0:h] = jnp.stack(c1, axis=0)
        e2_scr[tile_idx, 1:T, 0:h] = jnp.stack(c2, axis=0)
        e4_scr[tile_idx, 3:T, 0:h] = jnp.stack(c4, axis=0)

    def tile_body(i, carry):
        do_tile(pl.ds(i * TILE, TILE), i, TILE)
        return carry

    jax.lax.fori_loop(0, N_FULL, tile_body, 0)
    do_tile(slice(N_FULL * TILE, N_TOK), N_FULL, REM)

    for t in range(T):
        p_sum = jnp.dot(oh, p_scr[t], preferred_element_type=jnp.float32)
        p_reg = p_sum / (cnt + EPS)
        hr_scr[t, :] = _h(p_reg)

    vm = vm_ref[...] > 0.5
    vm3 = jnp.broadcast_to(vm, (N_TILES, T, TILE))
    e1 = e1_scr[...]
    i2 = _interp_t(e2_scr[:, 1:T, :], I2_COEF)
    i4 = _interp_t(e4_scr[:, 3:T, :], I4_COEF)
    ht = (e1 + i2 + i4) * (1.0 / 3.0)
    mn = jnp.min(jnp.where(vm3, ht, 1e30))
    mx = jnp.max(jnp.where(vm3, ht, -1e30))
    ht_n = (ht - mn) / (mx - mn + EPS)
    hr = hr_scr[...]

    def _nrm(hsub):
        return (hsub - jnp.min(hsub)) / (jnp.max(hsub) - jnp.min(hsub) + EPS)

    hc_n = _nrm(hr[:, :R_C])
    hf_n = _nrm(hr[:, R_C:])
    ohp = ohp_ref[...]
    hc_tok = sum(hc_n[:, r].reshape(1, T, 1) * ohp[r][:, None, :]
                 for r in range(R_C))
    hf_tok = sum(hf_n[:, r].reshape(1, T, 1) * ohp[R_C + r][:, None, :]
                 for r in range(R_F))
    score = ALPHA * ht_n + BETA * hc_tok + GAMMA * hf_tok
    score_m = jnp.where(vm3, score, -1.0)
    kth = _kth_largest(score_m, K_TOP)
    mask = jax.nn.sigmoid((score - kth) * (1.0 / TOK_TEMP))
    mask_ref[0] = mask

    prev = jnp.where(b == 0, 0.0, tot_scr[...][0, 0])
    total = prev + jnp.sum(jnp.where(vm3, mask, 0.0))
    tot_scr[...] = jnp.reshape(total, (1, 1))

    head_w = jax.nn.sigmoid(ghead_ref[...])
    ch_w = jax.nn.sigmoid(gch_ref[...])
    block_w = jax.nn.sigmoid(gblock_ref[...])
    headw_ref[...] = head_w
    chw_ref[...] = ch_w
    blockw_ref[...] = block_w
    sparsity_token = 1.0 - total / float(B * T * N_TOK)
    l_ast = (sparsity_token + (1.0 - jnp.mean(head_w))
             + (1.0 - jnp.mean(ch_w)) + (1.0 - jnp.mean(block_w)))
    st_ref[...] = jnp.reshape(sparsity_token, (1, 1))
    last_ref[...] = jnp.reshape(l_ast, (1, 1))


def kernel(token_feat, centers_coarse, centers_fine, g_head, g_ch, g_block,
           patch_coords):
    oh = jnp.concatenate([
        _region_one_hot(patch_coords, centers_coarse),
        _region_one_hot(patch_coords, centers_fine),
    ], axis=0)
    oh_pad = jnp.pad(oh, ((0, 0), (0, N_PAD - N_TOK))).reshape(
        R_C + R_F, N_TILES, TILE)
    vmask = (np.arange(N_PAD).reshape(N_TILES, 1, TILE) < N_TOK
             ).astype(np.float32)

    n, c = N_TOK, EMBED_DIM
    const = lambda b: (0, 0)
    mask4, head_w, ch_w, block_w2, st, last = pl.pallas_call(
        _fused_kernel,
        grid=(B,),
        in_specs=[
            pl.BlockSpec((1, T, n, c), lambda b: (b, 0, 0, 0)),
            pl.BlockSpec((R_C + R_F, n), const),
            pl.BlockSpec((R_C + R_F, N_TILES, TILE), lambda b: (0, 0, 0)),
            pl.BlockSpec((N_TILES, 1, TILE), lambda b: (0, 0, 0)),
            pl.BlockSpec((DEPTH, NUM_HEADS), const),
            pl.BlockSpec((DEPTH, HIDDEN_DIM), const),
            pl.BlockSpec((1, DEPTH), const),
        ],
        out_specs=[
            pl.BlockSpec((1, N_TILES, T, TILE), lambda b: (b, 0, 0, 0)),
            pl.BlockSpec((DEPTH, NUM_HEADS), const),
            pl.BlockSpec((DEPTH, HIDDEN_DIM), const),
            pl.BlockSpec((1, DEPTH), const),
            pl.BlockSpec((1, 1), const),
            pl.BlockSpec((1, 1), const),
        ],
        out_shape=[
            jax.ShapeDtypeStruct((B, N_TILES, T, TILE), jnp.float32),
            jax.ShapeDtypeStruct((DEPTH, NUM_HEADS), jnp.float32),
            jax.ShapeDtypeStruct((DEPTH, HIDDEN_DIM), jnp.float32),
            jax.ShapeDtypeStruct((1, DEPTH), jnp.float32),
            jax.ShapeDtypeStruct((1, 1), jnp.float32),
            jax.ShapeDtypeStruct((1, 1), jnp.float32),
        ],
        scratch_shapes=[
            pltpu.VMEM((N_TILES, T, TILE), jnp.float32),
            pltpu.VMEM((N_TILES, T, TILE), jnp.float32),
            pltpu.VMEM((N_TILES, T, TILE), jnp.float32),
            pltpu.VMEM((T, R_C + R_F), jnp.float32),
            pltpu.VMEM((1, 1), jnp.float32),
            pltpu.VMEM((T, n, c), jnp.float32),
        ],
    )(token_feat, oh, oh_pad, jnp.asarray(vmask),
      g_head, g_ch, g_block.reshape(1, DEPTH))
    mask = jnp.transpose(mask4, (0, 2, 1, 3)).reshape(B, T, N_PAD)[:, :, :n]
    return (mask, head_w, ch_w, block_w2.reshape(DEPTH),
            st.reshape(()), last.reshape(()))
